# baseline (speedup 1.0000x reference)
import jax
import jax.numpy as jnp
from jax import lax
from jax.experimental import pallas as pl
from jax.experimental.pallas import tpu as pltpu
from jax.experimental.pallas import tpu_sc as plsc


BATCH = 4
SEQ_LEN = 8192
DIM = 768
ROWS = BATCH * SEQ_LEN
NUM_CORES = 2
NUM_SUBCORES = 16
NW = NUM_CORES * NUM_SUBCORES
ROWS_PER_W = ROWS // NW
CHUNK = 64
N_CHUNKS = ROWS_PER_W // CHUNK
W_PER_BATCH = SEQ_LEN // ROWS_PER_W


def _sc_body(x_hbm, pos_hbm, out_hbm, xbuf, posbuf, idxbuf, semx, semg):
    cid = lax.axis_index("c")
    sid = lax.axis_index("s")
    wid = sid * NUM_CORES + cid
    base = wid * ROWS_PER_W
    t_base = lax.rem(base, SEQ_LEN)

    def chunk_step(c, carry):
        r0 = base + c * CHUNK
        t0 = t_base + c * CHUNK
        cp_x = pltpu.async_copy(x_hbm.at[pl.ds(r0, CHUNK)], xbuf, semx)
        for j in range(CHUNK // 16):
            idxbuf[pl.ds(j * 16, 16)] = (
                t0 + j * 16 + lax.broadcasted_iota(jnp.int32, (16,), 0)
            )
        cp_g = pltpu.async_copy(pos_hbm.at[idxbuf], posbuf, semg)
        cp_x.wait()
        cp_g.wait()

        def row_add(r, acc):
            for j in range(DIM // 16):
                plsc.addupdate(
                    xbuf.at[r, pl.ds(j * 16, 16)],
                    posbuf[r, pl.ds(j * 16, 16)],
                )
            return acc

        lax.fori_loop(0, CHUNK, row_add, 0)
        pltpu.sync_copy(xbuf, out_hbm.at[pl.ds(r0, CHUNK)])
        return carry

    lax.fori_loop(0, N_CHUNKS, chunk_step, 0)


def _sc_kernel(x, position_embed):
    x2 = x.reshape(ROWS, DIM)
    mesh = plsc.VectorSubcoreMesh(
        core_axis_name="c", subcore_axis_name="s",
        num_cores=NUM_CORES, num_subcores=NUM_SUBCORES,
    )
    out = pl.kernel(
        _sc_body,
        out_type=jax.ShapeDtypeStruct((ROWS, DIM), jnp.float32),
        mesh=mesh,
        scratch_types=[
            pltpu.VMEM((CHUNK, DIM), jnp.float32),
            pltpu.VMEM((CHUNK, DIM), jnp.float32),
            pltpu.VMEM((CHUNK,), jnp.int32),
            pltpu.SemaphoreType.DMA,
            pltpu.SemaphoreType.DMA,
        ],
    )(x2, position_embed)
    return out.reshape(BATCH, SEQ_LEN, DIM)


_SEQ_BLOCK = 512


def _tc_add_body(x_ref, pos_ref, out_ref):
    out_ref[...] = x_ref[...] + pos_ref[...][None, :, :]


def _tc_kernel(x, position_embed):
    batch, seq_len, dim = x.shape
    grid = (seq_len // _SEQ_BLOCK,)
    return pl.pallas_call(
        _tc_add_body,
        grid=grid,
        in_specs=[
            pl.BlockSpec((batch, _SEQ_BLOCK, dim), lambda i: (0, i, 0)),
            pl.BlockSpec((_SEQ_BLOCK, dim), lambda i: (i, 0)),
        ],
        out_specs=pl.BlockSpec((batch, _SEQ_BLOCK, dim), lambda i: (0, i, 0)),
        out_shape=jax.ShapeDtypeStruct(x.shape, x.dtype),
        compiler_params=pltpu.CompilerParams(
            dimension_semantics=("parallel",),
        ),
    )(x, position_embed)


def kernel(x, position_embed):
    return _sc_kernel(x, position_embed)


# speedup vs baseline: 1.3657x; 1.3657x over previous
import jax
import jax.numpy as jnp
from jax import lax
from jax.experimental import pallas as pl
from jax.experimental.pallas import tpu as pltpu
from jax.experimental.pallas import tpu_sc as plsc


BATCH = 4
SEQ_LEN = 8192
DIM = 768
ROWS = BATCH * SEQ_LEN
NUM_CORES = 2
NUM_SUBCORES = 16
NW = NUM_CORES * NUM_SUBCORES
ROWS_PER_W = ROWS // NW
CHUNK = 32
N_CHUNKS = ROWS_PER_W // CHUNK
W_PER_BATCH = SEQ_LEN // ROWS_PER_W


def _sc_body(x_hbm, pos_hbm, out_hbm,
             xbufs, posbufs, idxbufs, semx, semg, semo):
    cid = lax.axis_index("c")
    sid = lax.axis_index("s")
    wid = sid * NUM_CORES + cid
    base = wid * ROWS_PER_W
    t_base = lax.rem(base, SEQ_LEN)

    def start_in(c, b):
        r0 = base + c * CHUNK
        t0 = t_base + c * CHUNK
        for j in range(CHUNK // 16):
            idxbufs[b][pl.ds(j * 16, 16)] = (
                t0 + j * 16 + lax.broadcasted_iota(jnp.int32, (16,), 0)
            )
        pltpu.async_copy(x_hbm.at[pl.ds(r0, CHUNK)], xbufs[b], semx[b])
        pltpu.async_copy(pos_hbm.at[idxbufs[b]], posbufs[b], semg[b])

    def wait_in(b):
        pltpu.make_async_copy(x_hbm.at[pl.ds(0, CHUNK)], xbufs[b], semx[b]).wait()
        pltpu.make_async_copy(pos_hbm.at[idxbufs[b]], posbufs[b], semg[b]).wait()

    def add_chunk(b):
        def row_add(r, acc):
            for j in range(DIM // 16):
                plsc.addupdate(
                    xbufs[b].at[r, pl.ds(j * 16, 16)],
                    posbufs[b][r, pl.ds(j * 16, 16)],
                )
            return acc
        lax.fori_loop(0, CHUNK, row_add, 0)

    def start_out(c, b):
        r0 = base + c * CHUNK
        pltpu.async_copy(xbufs[b], out_hbm.at[pl.ds(r0, CHUNK)], semo[b])

    def wait_out(b):
        pltpu.make_async_copy(
            xbufs[b], out_hbm.at[pl.ds(0, CHUNK)], semo[b]).wait()

    start_in(0, 0)

    def pair_step(p, carry):
        c0 = 2 * p
        @pl.when(p > 0)
        def _():
            wait_out(1)
        start_in(c0 + 1, 1)
        wait_in(0)
        add_chunk(0)
        start_out(c0, 0)
        wait_in(1)
        add_chunk(1)
        start_out(c0 + 1, 1)
        @pl.when(p < N_CHUNKS // 2 - 1)
        def _():
            wait_out(0)
            start_in(c0 + 2, 0)
        return carry

    lax.fori_loop(0, N_CHUNKS // 2, pair_step, 0)
    wait_out(0)
    wait_out(1)


def _sc_kernel(x, position_embed):
    x2 = x.reshape(ROWS, DIM)
    mesh = plsc.VectorSubcoreMesh(
        core_axis_name="c", subcore_axis_name="s",
        num_cores=NUM_CORES, num_subcores=NUM_SUBCORES,
    )
    out = pl.kernel(
        _sc_body,
        out_type=jax.ShapeDtypeStruct((ROWS, DIM), jnp.float32),
        mesh=mesh,
        scratch_types=[
            [pltpu.VMEM((CHUNK, DIM), jnp.float32) for _ in range(2)],
            [pltpu.VMEM((CHUNK, DIM), jnp.float32) for _ in range(2)],
            [pltpu.VMEM((CHUNK,), jnp.int32) for _ in range(2)],
            [pltpu.SemaphoreType.DMA for _ in range(2)],
            [pltpu.SemaphoreType.DMA for _ in range(2)],
            [pltpu.SemaphoreType.DMA for _ in range(2)],
        ],
    )(x2, position_embed)
    return out.reshape(BATCH, SEQ_LEN, DIM)


_SEQ_BLOCK = 512


def _tc_add_body(x_ref, pos_ref, out_ref):
    out_ref[...] = x_ref[...] + pos_ref[...][None, :, :]


def _tc_kernel(x, position_embed):
    batch, seq_len, dim = x.shape
    grid = (seq_len // _SEQ_BLOCK,)
    return pl.pallas_call(
        _tc_add_body,
        grid=grid,
        in_specs=[
            pl.BlockSpec((batch, _SEQ_BLOCK, dim), lambda i: (0, i, 0)),
            pl.BlockSpec((_SEQ_BLOCK, dim), lambda i: (i, 0)),
        ],
        out_specs=pl.BlockSpec((batch, _SEQ_BLOCK, dim), lambda i: (0, i, 0)),
        out_shape=jax.ShapeDtypeStruct(x.shape, x.dtype),
        compiler_params=pltpu.CompilerParams(
            dimension_semantics=("parallel",),
        ),
    )(x, position_embed)


def kernel(x, position_embed):
    return _sc_kernel(x, position_embed)


# speedup vs baseline: 1.6790x; 1.2294x over previous
import jax
import jax.numpy as jnp
from jax import lax
from jax.experimental import pallas as pl
from jax.experimental.pallas import tpu as pltpu
from jax.experimental.pallas import tpu_sc as plsc


BATCH = 4
SEQ_LEN = 8192
DIM = 768
ROWS = BATCH * SEQ_LEN
NUM_CORES = 2
NUM_SUBCORES = 16
NW = NUM_CORES * NUM_SUBCORES
ROWS_PER_W = ROWS // NW
CHUNK = 32
N_CHUNKS = ROWS_PER_W // CHUNK
W_PER_BATCH = SEQ_LEN // ROWS_PER_W


T_PER_W = SEQ_LEN // NW
TCH = 16
NCH = T_PER_W // TCH
STAGES = NCH * BATCH
NXB = 4
PREF = 2


def _sc_body(x_hbm, pos_hbm, out_hbm, xbufs, posbufs, semp, semx, semo):
    cid = lax.axis_index("c")
    sid = lax.axis_index("s")
    wid = sid * NUM_CORES + cid
    t_base = wid * T_PER_W

    def start_pos(c, slot):
        pltpu.async_copy(
            pos_hbm.at[pl.ds(t_base + c * TCH, TCH)], posbufs[slot],
            semp[slot])

    def wait_pos(slot):
        pltpu.make_async_copy(
            pos_hbm.at[pl.ds(0, TCH)], posbufs[slot], semp[slot]).wait()

    def start_x(c, b, slot):
        r0 = b * SEQ_LEN + t_base + c * TCH
        pltpu.async_copy(x_hbm.at[pl.ds(r0, TCH)], xbufs[slot], semx[slot])

    def wait_x(slot):
        pltpu.make_async_copy(
            x_hbm.at[pl.ds(0, TCH)], xbufs[slot], semx[slot]).wait()

    def start_out(c, b, slot):
        r0 = b * SEQ_LEN + t_base + c * TCH
        pltpu.async_copy(xbufs[slot], out_hbm.at[pl.ds(r0, TCH)], semo[slot])

    def wait_out(slot):
        pltpu.make_async_copy(
            xbufs[slot], out_hbm.at[pl.ds(0, TCH)], semo[slot]).wait()

    def add_stage(slot, pslot):
        xb, pb = xbufs[slot], posbufs[pslot]

        def row_add(r, acc):
            for j in range(DIM // 16):
                plsc.addupdate(
                    xb.at[r, pl.ds(j * 16, 16)], pb[r, pl.ds(j * 16, 16)])
            return acc
        lax.fori_loop(0, TCH, row_add, 0)

    def stage_body(oct_, u, first_octet=False, last_octet=False):
        c = 2 * oct_ + u // 4
        b = u % 4
        pslot = (u // 4) % 2
        if b == 0:
            wait_pos(pslot)
        if b == 1 and not (last_octet and u == 5):
            start_pos(c + 1, (pslot + 1) % 2)
        wait_x(u % 4)
        add_stage(u % 4, pslot)
        start_out(c, b, u % 4)
        if not (last_octet and u >= 8 - PREF):
            if not (first_octet and u < PREF):
                wait_out((u + PREF) % NXB)
            off = u + PREF
            start_x(2 * oct_ + off // 4, off % 4, off % NXB)

    n_oct = STAGES // 8
    start_pos(0, 0)
    start_x(0, 0, 0)
    start_x(0, 1, 1)
    for u in range(8):
        stage_body(0, u, first_octet=True)

    def octet(q, carry):
        for u in range(8):
            stage_body(q, u)
        return carry

    lax.fori_loop(1, n_oct - 1, octet, 0)
    for u in range(8):
        stage_body(n_oct - 1, u, last_octet=True)
    for slot in range(NXB):
        wait_out(slot)


def _sc_kernel(x, position_embed):
    x2 = x.reshape(ROWS, DIM)
    mesh = plsc.VectorSubcoreMesh(
        core_axis_name="c", subcore_axis_name="s",
        num_cores=NUM_CORES, num_subcores=NUM_SUBCORES,
    )
    out = pl.kernel(
        _sc_body,
        out_type=jax.ShapeDtypeStruct((ROWS, DIM), jnp.float32),
        mesh=mesh,
        scratch_types=[
            [pltpu.VMEM((TCH, DIM), jnp.float32) for _ in range(NXB)],
            [pltpu.VMEM((TCH, DIM), jnp.float32) for _ in range(2)],
            [pltpu.SemaphoreType.DMA for _ in range(2)],
            [pltpu.SemaphoreType.DMA for _ in range(NXB)],
            [pltpu.SemaphoreType.DMA for _ in range(NXB)],
        ],
    )(x2, position_embed)
    return out.reshape(BATCH, SEQ_LEN, DIM)


_SEQ_BLOCK = 512


def _tc_add_body(x_ref, pos_ref, out_ref):
    out_ref[...] = x_ref[...] + pos_ref[...][None, :, :]


def _tc_kernel(x, position_embed):
    batch, seq_len, dim = x.shape
    grid = (seq_len // _SEQ_BLOCK,)
    return pl.pallas_call(
        _tc_add_body,
        grid=grid,
        in_specs=[
            pl.BlockSpec((batch, _SEQ_BLOCK, dim), lambda i: (0, i, 0)),
            pl.BlockSpec((_SEQ_BLOCK, dim), lambda i: (i, 0)),
        ],
        out_specs=pl.BlockSpec((batch, _SEQ_BLOCK, dim), lambda i: (0, i, 0)),
        out_shape=jax.ShapeDtypeStruct(x.shape, x.dtype),
        compiler_params=pltpu.CompilerParams(
            dimension_semantics=("parallel",),
        ),
    )(x, position_embed)


def kernel(x, position_embed):
    return _sc_kernel(x, position_embed)


# speedup vs baseline: 2.0212x; 1.2038x over previous
import jax
import jax.numpy as jnp
from jax import lax
from jax.experimental import pallas as pl
from jax.experimental.pallas import tpu as pltpu
from jax.experimental.pallas import tpu_sc as plsc


BATCH = 4
SEQ_LEN = 8192
DIM = 768
ROWS = BATCH * SEQ_LEN
NUM_CORES = 2
NUM_SUBCORES = 16
NW = NUM_CORES * NUM_SUBCORES
ROWS_PER_W = ROWS // NW
CHUNK = 32
N_CHUNKS = ROWS_PER_W // CHUNK
W_PER_BATCH = SEQ_LEN // ROWS_PER_W


T_PER_W = SEQ_LEN // NW
TCH = 16
NCH = T_PER_W // TCH
STAGES = NCH * BATCH
NXB = 8
PREF = 4
_PROBE_NO_ADD = False


def _sc_body(x_hbm, pos_hbm, out_hbm, xbufs, posbufs, semp, semx, semo):
    cid = lax.axis_index("c")
    sid = lax.axis_index("s")
    wid = sid * NUM_CORES + cid
    t_base = wid * T_PER_W

    def start_pos(c, slot):
        pltpu.async_copy(
            pos_hbm.at[pl.ds(t_base + c * TCH, TCH)], posbufs[slot],
            semp[slot])

    def wait_pos(slot):
        pltpu.make_async_copy(
            pos_hbm.at[pl.ds(0, TCH)], posbufs[slot], semp[slot]).wait()

    def start_x(c, b, slot):
        r0 = b * SEQ_LEN + t_base + c * TCH
        pltpu.async_copy(x_hbm.at[pl.ds(r0, TCH)], xbufs[slot], semx[slot])

    def wait_x(slot):
        pltpu.make_async_copy(
            x_hbm.at[pl.ds(0, TCH)], xbufs[slot], semx[slot]).wait()

    def start_out(c, b, slot):
        r0 = b * SEQ_LEN + t_base + c * TCH
        pltpu.async_copy(xbufs[slot], out_hbm.at[pl.ds(r0, TCH)], semo[slot])

    def wait_out(slot):
        pltpu.make_async_copy(
            xbufs[slot], out_hbm.at[pl.ds(0, TCH)], semo[slot]).wait()

    def add_stage(slot, pslot):
        xb, pb = xbufs[slot], posbufs[pslot]

        def row_add(r, acc):
            for j in range(DIM // 16):
                plsc.addupdate(
                    xb.at[r, pl.ds(j * 16, 16)], pb[r, pl.ds(j * 16, 16)])
            return acc
        if _PROBE_NO_ADD:
            return
        lax.fori_loop(0, TCH, row_add, 0)

    def stage_body(oct_, u, first_octet=False, last_octet=False):
        c = 2 * oct_ + u // 4
        b = u % 4
        pslot = (u // 4) % 2
        if b == 0:
            wait_pos(pslot)
        if b == 1 and not (last_octet and u == 5):
            start_pos(c + 1, (pslot + 1) % 2)
        wait_x(u % NXB)
        add_stage(u % NXB, pslot)
        start_out(c, b, u % NXB)
        if not (last_octet and u >= 8 - PREF):
            if not (first_octet and u < PREF):
                wait_out((u + PREF) % NXB)
            off = u + PREF
            start_x(2 * oct_ + off // 4, off % 4, off % NXB)

    n_oct = STAGES // 8
    start_pos(0, 0)
    for u in range(PREF):
        start_x(u // 4, u % 4, u % NXB)
    for u in range(8):
        stage_body(0, u, first_octet=True)

    def octet(q, carry):
        for u in range(8):
            stage_body(q, u)
        return carry

    lax.fori_loop(1, n_oct - 1, octet, 0)
    for u in range(8):
        stage_body(n_oct - 1, u, last_octet=True)
    for slot in range(NXB):
        wait_out(slot)


def _sc_kernel(x, position_embed):
    x2 = x.reshape(ROWS, DIM)
    mesh = plsc.VectorSubcoreMesh(
        core_axis_name="c", subcore_axis_name="s",
        num_cores=NUM_CORES, num_subcores=NUM_SUBCORES,
    )
    out = pl.kernel(
        _sc_body,
        out_type=jax.ShapeDtypeStruct((ROWS, DIM), jnp.float32),
        mesh=mesh,
        scratch_types=[
            [pltpu.VMEM((TCH, DIM), jnp.float32) for _ in range(NXB)],
            [pltpu.VMEM((TCH, DIM), jnp.float32) for _ in range(2)],
            [pltpu.SemaphoreType.DMA for _ in range(2)],
            [pltpu.SemaphoreType.DMA for _ in range(NXB)],
            [pltpu.SemaphoreType.DMA for _ in range(NXB)],
        ],
    )(x2, position_embed)
    return out.reshape(BATCH, SEQ_LEN, DIM)


_SEQ_BLOCK = 512


def _tc_add_body(x_ref, pos_ref, out_ref):
    out_ref[...] = x_ref[...] + pos_ref[...][None, :, :]


def _tc_kernel(x, position_embed):
    batch, seq_len, dim = x.shape
    grid = (seq_len // _SEQ_BLOCK,)
    return pl.pallas_call(
        _tc_add_body,
        grid=grid,
        in_specs=[
            pl.BlockSpec((batch, _SEQ_BLOCK, dim), lambda i: (0, i, 0)),
            pl.BlockSpec((_SEQ_BLOCK, dim), lambda i: (i, 0)),
        ],
        out_specs=pl.BlockSpec((batch, _SEQ_BLOCK, dim), lambda i: (0, i, 0)),
        out_shape=jax.ShapeDtypeStruct(x.shape, x.dtype),
        compiler_params=pltpu.CompilerParams(
            dimension_semantics=("parallel",),
        ),
    )(x, position_embed)


def kernel(x, position_embed):
    return _sc_kernel(x, position_embed)
